# baseline (speedup 1.0000x reference)
import functools

import jax
import jax.numpy as jnp
from jax import lax
from jax.experimental import pallas as pl
from jax.experimental.pallas import tpu as pltpu
from jax.experimental.pallas import tpu_sc as plsc

jax.config.update("jax_enable_x64", True)

_LEVELS = 100
_TS = 512
_D = 10000
_N = 512
_W = 1024
_GRID = -(-_D // _W)
_NW = 32
_RPW = _N // _NW
_CH = 8
_DSC = 10112
_VECS = _DSC // 16
_FEAT_ORDER = [558, 582, 554, 552, 93, 555, 580, 571, 574, 578, 566, 287,
               556, 550, 14, 551, 64, 581]
_ROW = {k: i for i, k in enumerate(_FEAT_ORDER)}


def _sc_bag_body(lt_hbm, idx_hbm, out_hbm, idx_v, rows_v, acc_v, sem):
    i32 = jnp.int32
    wid = (lax.axis_index("s").astype(i32) * i32(2)
           + lax.axis_index("c").astype(i32))
    base = pl.multiple_of(wid * i32(_RPW), _RPW)
    pltpu.sync_copy(idx_hbm.at[pl.ds(base, _RPW)], idx_v)

    for c in range(_RPW // _CH):
        pltpu.async_copy(lt_hbm.at[idx_v.at[pl.ds(c * _CH, _CH)]],
                         rows_v, sem).wait()

        def add_rows(j, off):
            sl = pl.ds(pl.multiple_of(off, 16), 16)
            t = rows_v[0, sl]
            for r in range(1, _CH):
                t = t + rows_v[r, sl]
            if c == 0:
                acc_v[sl] = t
            else:
                acc_v[sl] = acc_v[sl] + t
            return off + i32(16)

        lax.fori_loop(0, _VECS, add_rows, i32(0), unroll=2)

    pltpu.sync_copy(acc_v, out_hbm.at[wid])


@functools.partial(
    pl.kernel,
    mesh=plsc.VectorSubcoreMesh(core_axis_name="c", subcore_axis_name="s"),
    out_type=jax.ShapeDtypeStruct((_NW, _DSC), jnp.float32),
    scratch_types=[
        pltpu.VMEM((_RPW,), jnp.int32),
        pltpu.VMEM((_CH, _DSC), jnp.float32),
        pltpu.VMEM((_DSC,), jnp.float32),
        pltpu.SemaphoreType.DMA,
    ],
)
def _sc_bag(lt_hbm, idx_hbm, out_hbm, idx_v, rows_v, acc_v, sem):
    _sc_bag_body(lt_hbm, idx_hbm, out_hbm, idx_v, rows_v, acc_v, sem)


def _tile_body(idx_ref, fv_ref, sxyz_ref, st_ref, sw_ref, sb_ref, out_ref):
    f32 = jnp.float32

    ix = idx_ref[:, 0:1]
    iy = idx_ref[:, 1:2]
    iz = idx_ref[:, 2:3]

    iota_xyz = lax.broadcasted_iota(jnp.int32, (_N, 304), 1)
    oh = ((iota_xyz == ix) | (iota_xyz == iy) | (iota_xyz == iz))
    oh = oh.astype(jnp.bfloat16)
    s = lax.dot_general(oh, sxyz_ref[...].astype(jnp.bfloat16),
                        (((1,), (0,)), ((), ())),
                        preferred_element_type=f32)

    neg_s = jnp.sum((s < 0).astype(f32), axis=0, keepdims=True)
    cnt3 = jnp.sum((jnp.abs(s) > 2.0).astype(f32), axis=0, keepdims=True)

    sum_t = jnp.sum(st_ref[...], axis=0, keepdims=True)
    neg_t = (512.0 - sum_t) * 0.5

    m = neg_s + neg_t
    parity = m - 2.0 * jnp.floor(m * 0.5)
    sign = 1.0 - 2.0 * parity

    c = lambda v: jnp.float32(v)
    pow3 = jnp.where(cnt3 == 0.0, c(1.0),
           jnp.where(cnt3 == 1.0, c(3.0),
           jnp.where(cnt3 == 2.0, c(9.0),
           jnp.where(cnt3 == 3.0, c(27.0), c(1e6)))))
    sample_hv = sign * pow3

    fv = fv_ref[:, 0:1]
    proj = fv * sw_ref[...]
    f = jnp.cos(proj + sb_ref[...]) * jnp.sin(proj)

    def r(k):
        i = _ROW[k]
        return f[i:i + 1, :]

    feat_hv = ((r(14) + r(287)) * r(64)
               * (r(93) + r(574) + r(580) + r(582) + r(555) + r(556) + r(581))
               * r(550) * (r(551) + r(554)) * r(552) * r(558) * r(566)
               * r(571) * r(578))

    combined = sample_hv + feat_hv
    quant = jnp.where(combined > 0.0, jnp.float32(1.0), jnp.float32(-1.0))
    out_ref[...] = jnp.where(cnt3 > 80.5, jnp.float32(-1.0), quant)


def _im_fixed(j):
    z = jnp.asarray(0, jnp.int32)
    return (z, z)


def _im_tile(j):
    return (jnp.asarray(0, jnp.int32), jnp.asarray(j, jnp.int32))


@jax.jit
def kernel(input, feat, level_x, level_y, level_z, level_t, sin_w, sin_b):
    f64 = jnp.float64

    def lookup_idx(value, low, high, num):
        idx = jnp.round((value - low) / (high - low) * (num - 1))
        return jnp.clip(idx, 0.0, float(num - 1)).astype(jnp.int32)

    x_sig = jnp.clip(input[:, 1], -5.0, 5.0)
    y_sig = jnp.clip(input[:, 2], -5.0, 5.0)
    z_sig = jnp.clip(input[:, 3], -5.0, 5.0)
    ix = lookup_idx(x_sig, -5.0, 5.0, _LEVELS)
    iy = lookup_idx(y_sig, -5.0, 5.0, _LEVELS) + 100
    iz = lookup_idx(z_sig, -5.0, 5.0, _LEVELS) + 200
    it = lookup_idx(input[:, 0], 0.0, float(_TS), _TS)

    idx_cols = jnp.zeros((_N, 128), jnp.int32)
    idx_cols = idx_cols.at[:, 0].set(ix).at[:, 1].set(iy)
    idx_cols = idx_cols.at[:, 2].set(iz)

    fvals = feat[jnp.array(_FEAT_ORDER)].astype(jnp.float32)
    fv = jnp.zeros((24, 128), jnp.float32).at[:18, :].set(fvals[:, None])

    sxyz = jnp.concatenate([level_x.astype(jnp.float32),
                            level_y.astype(jnp.float32),
                            level_z.astype(jnp.float32)], axis=0)
    sxyz = jnp.pad(sxyz, ((0, 4), (0, 0)))
    lt = jnp.pad(level_t.astype(jnp.float32),
                 ((0, 0), (0, _DSC - _D)))
    sw = jnp.pad(sin_w[:, :, 0].astype(jnp.float32), ((0, 6), (0, 0)))
    sb = jnp.pad(sin_b[:, 0, :].astype(jnp.float32), ((0, 6), (0, 0)))

    sum_t_parts = _sc_bag(lt, it)

    out = pl.pallas_call(
        _tile_body,
        grid=(_GRID,),
        in_specs=[
            pl.BlockSpec((_N, 128), _im_fixed),
            pl.BlockSpec((24, 128), _im_fixed),
            pl.BlockSpec((304, _W), _im_tile),
            pl.BlockSpec((_NW, _W), _im_tile),
            pl.BlockSpec((24, _W), _im_tile),
            pl.BlockSpec((24, _W), _im_tile),
        ],
        out_specs=pl.BlockSpec((1, _W), _im_tile),
        out_shape=jax.ShapeDtypeStruct((1, _D), jnp.float32),
    )(idx_cols, fv, sxyz, sum_t_parts, sw, sb)

    return out[0, :].astype(f64)
